# speedup vs baseline: 7.2235x; 7.2235x over previous
import jax
import jax.numpy as jnp
from jax.experimental import pallas as pl
from jax.experimental.pallas import tpu as pltpu

N_AGENT = 25000
N_TRACK = 25000
HID = 64
HEADS = [4, 4, 2]
_BN = 1.0 / (1.0 + 1e-5) ** 0.5


def _postproc_body(p00, p01, b, res, o):
    x = jax.nn.relu(p00[...] + p01[...] + b[...]) * _BN
    o[...] = x + res[...]


def _postproc(p0, p1, bias, res):
    n = p0.shape[0]
    bm = 1000
    b2 = jnp.broadcast_to(bias[None, :], (1, HID))
    return pl.pallas_call(
        _postproc_body,
        grid=(n // bm,),
        in_specs=[
            pl.BlockSpec((bm, HID), lambda i: (i, 0)),
            pl.BlockSpec((bm, HID), lambda i: (i, 0)),
            pl.BlockSpec((1, HID), lambda i: (0, 0)),
            pl.BlockSpec((bm, HID), lambda i: (i, 0)),
        ],
        out_specs=pl.BlockSpec((bm, HID), lambda i: (i, 0)),
        out_shape=jax.ShapeDtypeStruct((n, HID), jnp.float32),
    )(p0, p1, b2, res)


def _gat(x_src, x_dst, ei, p, heads, num_dst):
    W, a_s, a_d, b = p
    src, dst = ei[0], ei[1]
    hs = (x_src @ W).reshape(-1, heads, HID)
    hd = (x_dst @ W).reshape(-1, heads, HID)
    als = (hs * a_s[None, :, :]).sum(-1)
    ald = (hd * a_d[None, :, :]).sum(-1)
    alpha = jax.nn.leaky_relu(als[src] + ald[dst], 0.2)
    ex = jnp.exp(alpha)
    den = jax.ops.segment_sum(ex, dst, num_segments=num_dst)
    coef = ex / (den[dst] + 1e-16)
    msg = (hs[src] * coef[:, :, None]).mean(axis=1)
    out = jax.ops.segment_sum(msg, dst, num_segments=num_dst)
    return out, b


def _hetero(xa, xt, e1, e2, e3, e4, ps, heads, ra, rt):
    o1, b1 = _gat(xa, xt, e1, ps[0], heads, N_TRACK)
    o3, b3 = _gat(xa, xt, e3, ps[2], heads, N_TRACK)
    o2, b2 = _gat(xt, xa, e2, ps[1], heads, N_AGENT)
    o4, b4 = _gat(xt, xa, e4, ps[3], heads, N_AGENT)
    out_t = _postproc(o1, o3, b1 + b3, rt)
    out_a = _postproc(o2, o4, b2 + b4, ra)
    return out_a, out_t


def _ln(x):
    m = x.mean(axis=-1, keepdims=True)
    v = ((x - m) ** 2).mean(axis=-1, keepdims=True)
    return (x - m) / jnp.sqrt(v + 1e-5)


def _value_mlp(x, ps):
    h = jax.nn.relu(_ln(x @ ps[0][0] + ps[0][1]))
    h = jax.nn.relu(_ln(h @ ps[1][0] + ps[1][1]))
    h = jax.nn.relu(h @ ps[2][0] + ps[2][1])
    return h @ ps[3][0] + ps[3][1]


def _head_mlp(x, ps):
    h = jax.nn.relu(_ln(x @ ps[0][0] + ps[0][1]))
    return jax.nn.softplus(h @ ps[1][0] + ps[1][1]) + 1.0


def kernel(x_agent, x_track, ei1, ei2, ei3, ei4, params):
    xa = jax.nn.relu(x_agent @ params['emb_agent'][0] + params['emb_agent'][1])
    xt = jax.nn.relu(x_track @ params['emb_track'][0] + params['emb_track'][1])
    z_a = jnp.zeros((N_AGENT, HID), jnp.float32)
    z_t = jnp.zeros((N_TRACK, HID), jnp.float32)
    a1, t1 = _hetero(xa, xt, ei1, ei2, ei3, ei4, params['gat'][0], HEADS[0], z_a, z_t)
    a2, t2 = _hetero(a1, t1, ei1, ei2, ei3, ei4, params['gat'][1], HEADS[1], a1, t1)
    a3, t3 = _hetero(a2, t2, ei1, ei2, ei3, ei4, params['gat'][2], HEADS[2], a2, t2)
    vr = params['value_reg']
    av = _value_mlp(a3, params['agent_value'])
    av = av * (1.0 - vr) + vr * jnp.tanh(av)
    tv = _value_mlp(t3, params['track_value'])
    tv = tv * (1.0 - vr) + vr * jnp.tanh(tv)
    a_heads = [_head_mlp(a3, p) for p in params['agent_heads']]
    t_heads = [_head_mlp(t3, p) for p in params['track_heads']]
    return jnp.concatenate(a_heads + [av] + t_heads + [tv], axis=1)


# speedup vs baseline: 10.9188x; 1.5116x over previous
import functools

import jax
import jax.numpy as jnp
from jax.experimental import pallas as pl

N_AGENT = 25000
N_TRACK = 25000
HID = 64
HEADS = [4, 4, 2]
_BN = 1.0 / (1.0 + 1e-5) ** 0.5

NP = 25088
CHUNK = NP // 16


def _mm_body(x_ref, w_ref, b_ref, o_ref, *, act):
    y = jnp.dot(x_ref[...], w_ref[...], preferred_element_type=jnp.float32)
    y = y + b_ref[...]
    if act == "relu":
        y = jnp.maximum(y, 0.0)
    o_ref[...] = y


def _mm(x, w, b=None, act=None):
    n, k = x.shape
    m = w.shape[1]
    bm = CHUNK
    if b is None:
        b = jnp.zeros((m,), jnp.float32)
    return pl.pallas_call(
        functools.partial(_mm_body, act=act),
        grid=(n // bm,),
        in_specs=[
            pl.BlockSpec((bm, k), lambda i: (i, 0)),
            pl.BlockSpec((k, m), lambda i: (0, 0)),
            pl.BlockSpec((1, m), lambda i: (0, 0)),
        ],
        out_specs=pl.BlockSpec((bm, m), lambda i: (i, 0)),
        out_shape=jax.ShapeDtypeStruct((n, m), jnp.float32),
    )(x, w, b.reshape(1, m))


def _post_body(p0_ref, p1_ref, b_ref, r_ref, o_ref):
    y = p0_ref[...] + p1_ref[...] + b_ref[...]
    o_ref[...] = jnp.maximum(y, 0.0) * _BN + r_ref[...]


def _post_body_nores(p0_ref, p1_ref, b_ref, o_ref):
    y = p0_ref[...] + p1_ref[...] + b_ref[...]
    o_ref[...] = jnp.maximum(y, 0.0) * _BN


def _postproc(o1, o3, bias, res):
    n = o1.shape[0]
    bm = CHUNK
    args = [o1, o3, bias.reshape(1, HID)]
    specs = [pl.BlockSpec((bm, HID), lambda i: (i, 0)) for _ in range(2)]
    specs.append(pl.BlockSpec((1, HID), lambda i: (0, 0)))
    body = _post_body_nores
    if res is not None:
        args.append(res)
        specs.append(pl.BlockSpec((bm, HID), lambda i: (i, 0)))
        body = _post_body
    return pl.pallas_call(
        body,
        grid=(n // bm,),
        in_specs=specs,
        out_specs=pl.BlockSpec((bm, HID), lambda i: (i, 0)),
        out_shape=jax.ShapeDtypeStruct((n, HID), jnp.float32),
    )(*args)


def _ln(x):
    m = x.mean(axis=-1, keepdims=True)
    v = ((x - m) ** 2).mean(axis=-1, keepdims=True)
    return (x - m) / jnp.sqrt(v + 1e-5)


def _final_body(x_ref, *refs):
    o_ref = refs[-1]
    vr = refs[-2][...][0, 0]
    vps = refs[0:8]
    x = x_ref[...]
    h = jax.nn.relu(_ln(jnp.dot(x, vps[0][...]) + vps[1][...]))
    h = jax.nn.relu(_ln(jnp.dot(h, vps[2][...]) + vps[3][...]))
    h = jax.nn.relu(jnp.dot(h, vps[4][...]) + vps[5][...])
    v = jnp.dot(h, vps[6][...]) + vps[7][...]
    v = v * (1.0 - vr) + vr * jnp.tanh(v)
    cols = []
    for i in range(4):
        hw0, hb0, hw1, hb1 = refs[8 + 4 * i: 12 + 4 * i]
        hh = jax.nn.relu(_ln(jnp.dot(x, hw0[...]) + hb0[...]))
        cols.append(jax.nn.softplus(jnp.dot(hh, hw1[...]) + hb1[...]) + 1.0)
    o_ref[...] = jnp.concatenate(cols + [v], axis=1)


def _final(x, value_ps, head_ps, vr):
    n = x.shape[0]
    bm = CHUNK
    args = [x]
    specs = [pl.BlockSpec((bm, HID), lambda i: (i, 0))]
    for (w, b) in value_ps:
        args += [w, b.reshape(1, -1)]
        specs += [pl.BlockSpec(w.shape, lambda i: (0, 0)),
                  pl.BlockSpec((1, b.shape[0]), lambda i: (0, 0))]
    for hp in head_ps:
        for (w, b) in hp:
            args += [w, b.reshape(1, -1)]
            specs += [pl.BlockSpec(w.shape, lambda i: (0, 0)),
                      pl.BlockSpec((1, b.shape[0]), lambda i: (0, 0))]
    args.append(vr.reshape(1, 1))
    specs.append(pl.BlockSpec((1, 1), lambda i: (0, 0)))
    return pl.pallas_call(
        _final_body,
        grid=(n // bm,),
        in_specs=specs,
        out_specs=pl.BlockSpec((bm, 5), lambda i: (i, 0)),
        out_shape=jax.ShapeDtypeStruct((n, 5), jnp.float32),
    )(*args)


def _conv(xsrc, xdst, src, dst, p, h, num_dst):
    W, a_s, a_d, b = p
    Wr = W.reshape(HID, h, HID)
    Ws = jnp.einsum("dhk,hk->dh", Wr, a_s)
    Wd = jnp.einsum("dhk,hk->dh", Wr, a_d)
    pad = jnp.zeros((HID, 16 - h), jnp.float32)
    ha = _mm(xsrc, W)
    als = _mm(xsrc, jnp.concatenate([Ws, pad], axis=1))[:, :h]
    ald = _mm(xdst, jnp.concatenate([Wd, pad], axis=1))[:, :h]
    alpha = jax.nn.leaky_relu(als[src] + ald[dst], 0.2)
    ex = jnp.exp(alpha)
    den = jax.ops.segment_sum(ex, dst, num_segments=num_dst)
    coef = (ex / (den[dst] + 1e-16)) * (1.0 / h)
    hs = ha.reshape(-1, h, HID)[src]
    msg = jnp.einsum("ehd,eh->ed", hs, coef)
    out = jax.ops.segment_sum(msg, dst, num_segments=num_dst)
    return jnp.pad(out, ((0, NP - num_dst), (0, 0))), b


def kernel(x_agent, x_track, ei1, ei2, ei3, ei4, params):
    edges = [(ei[0].astype(jnp.int32), ei[1].astype(jnp.int32))
             for ei in (ei1, ei2, ei3, ei4)]

    xap = jnp.pad(x_agent, ((0, NP - N_AGENT), (0, 0)))
    xtp = jnp.pad(x_track, ((0, NP - N_TRACK), (0, 0)))
    xa = _mm(xap, params["emb_agent"][0], params["emb_agent"][1], act="relu")
    xt = _mm(xtp, params["emb_track"][0], params["emb_track"][1], act="relu")

    for l in range(3):
        h = HEADS[l]
        ps = params["gat"][l]
        o1, b1 = _conv(xa, xt, *edges[0], ps[0], h, N_TRACK)
        o3, b3 = _conv(xa, xt, *edges[2], ps[2], h, N_TRACK)
        o2, b2 = _conv(xt, xa, *edges[1], ps[1], h, N_AGENT)
        o4, b4 = _conv(xt, xa, *edges[3], ps[3], h, N_AGENT)
        res_t = None if l == 0 else xt
        res_a = None if l == 0 else xa
        xt = _postproc(o1, o3, b1 + b3, res_t)
        xa = _postproc(o2, o4, b2 + b4, res_a)

    aout = _final(xa, params["agent_value"], params["agent_heads"],
                  params["value_reg"])
    tout = _final(xt, params["track_value"], params["track_heads"],
                  params["value_reg"])
    return jnp.concatenate([aout[:N_AGENT], tout[:N_TRACK]], axis=1)
